# Initial kernel scaffold; baseline (speedup 1.0000x reference)
import functools

import jax
import jax.numpy as jnp
from jax import lax
from jax.experimental import pallas as pl
from jax.experimental.pallas import tpu as pltpu
from jax.experimental.pallas import tpu_sc as plsc

_L = 16


def _make_sc_kernel(B, V, D, C, NC, NS):
  NW = NC * NS
  bw = B // NW
  groups = bw // _L

  mesh = plsc.VectorSubcoreMesh(core_axis_name="c", subcore_axis_name="s")

  @functools.partial(
      pl.kernel,
      out_type=(
          jax.ShapeDtypeStruct((B, D), jnp.float32),
          jax.ShapeDtypeStruct((B, D), jnp.float32),
      ),
      mesh=mesh,
      scratch_types=[
          pltpu.VMEM((bw * C,), jnp.int32),
          pltpu.VMEM((bw,), jnp.int32),
          pltpu.VMEM((V, D), jnp.float32),
          pltpu.VMEM((48,), jnp.float32),
          pltpu.VMEM((bw, D), jnp.float32),
          pltpu.VMEM((bw, D), jnp.float32),
      ],
  )
  def k(x_hbm, x1_hbm, emb_hbm, wb_hbm, y_hbm, y1_hbm,
        x_v, x1_v, emb_v, wb_v, y_v, y1_v):
    wid = lax.axis_index("s") * NC + lax.axis_index("c")
    base = wid * bw

    pltpu.sync_copy(x_hbm.at[pl.ds(base * C, bw * C)], x_v)
    pltpu.sync_copy(x1_hbm.at[pl.ds(base, bw)], x1_v)
    pltpu.sync_copy(emb_hbm, emb_v)
    pltpu.sync_copy(wb_hbm, wb_v)

    dsplat = [jnp.full((_L,), d, jnp.int32) for d in range(D)]
    wsp = [[plsc.load_gather(wb_v, [jnp.full((_L,), o * (C * D) + kk, jnp.int32)])
            for kk in range(C * D)] for o in range(D)]
    bsp = [plsc.load_gather(wb_v, [jnp.full((_L,), C * D * D + o, jnp.int32)])
           for o in range(D)]
    lane = lax.iota(jnp.int32, _L)

    def group(g, carry):
      item = g * _L + lane
      x1g = plsc.load_gather(x1_v, [item])
      for d in range(D):
        y1d = plsc.load_gather(emb_v, [x1g, dsplat[d]])
        plsc.store_scatter(y1_v, [item, dsplat[d]], y1d)
      acc = [bsp[o] for o in range(D)]
      itemC = item * C
      for j in range(C):
        xj = plsc.load_gather(x_v, [itemC + j])
        for d in range(D):
          e = plsc.load_gather(emb_v, [xj, dsplat[d]])
          kk = j * D + d
          for o in range(D):
            acc[o] = acc[o] + wsp[o][kk] * e
      for o in range(D):
        plsc.store_scatter(y_v, [item, dsplat[o]], acc[o])
      return carry

    lax.fori_loop(0, groups, group, None)

    pltpu.sync_copy(y_v, y_hbm.at[pl.ds(base, bw), :])
    pltpu.sync_copy(y1_v, y1_hbm.at[pl.ds(base, bw), :])

  return k


def kernel(x, x1, emb, W, b):
  B, C = x.shape
  V, D = emb.shape
  info = plsc.get_sparse_core_info()
  NC, NS = info.num_cores, info.num_subcores
  wb = jnp.concatenate(
      [W.reshape(-1), b, jnp.zeros((48 - W.size - b.size,), jnp.float32)])
  k = _make_sc_kernel(B, V, D, C, NC, NS)
  y, y1 = k(x.reshape(-1), x1, emb, wb)
  return (y, y1)


# speedup vs baseline: 4.5238x; 4.5238x over previous
import functools

import jax
import jax.numpy as jnp
from jax import lax
from jax.experimental import pallas as pl
from jax.experimental.pallas import tpu as pltpu
from jax.experimental.pallas import tpu_sc as plsc

_L = 16


def _make_sc_kernel(B, V, D, C, NC, NS):
  NW = NC * NS
  bw = B // NW
  groups = bw // _L

  mesh = plsc.VectorSubcoreMesh(core_axis_name="c", subcore_axis_name="s")

  @functools.partial(
      pl.kernel,
      out_type=(
          jax.ShapeDtypeStruct((B * D,), jnp.float32),
          jax.ShapeDtypeStruct((B * D,), jnp.float32),
      ),
      mesh=mesh,
      compiler_params=pltpu.CompilerParams(needs_layout_passes=False),
      scratch_types=[
          pltpu.VMEM((bw * C,), jnp.int32),
          pltpu.VMEM((bw,), jnp.int32),
          pltpu.VMEM((V * D,), jnp.float32),
          pltpu.VMEM(((C * D + 1) * D * _L,), jnp.float32),
          pltpu.VMEM((bw * D,), jnp.float32),
          pltpu.VMEM((bw * D,), jnp.float32),
      ],
  )
  def k(x_hbm, x1_hbm, emb_hbm, wb_hbm, y_hbm, y1_hbm,
        x_v, x1_v, emb_v, wb_v, y_v, y1_v):
    wid = lax.axis_index("s") * NC + lax.axis_index("c")
    base = wid * bw

    pltpu.sync_copy(x_hbm.at[pl.ds(base * C, bw * C)], x_v)
    pltpu.sync_copy(x1_hbm.at[pl.ds(base, bw)], x1_v)
    pltpu.sync_copy(emb_hbm, emb_v)
    pltpu.sync_copy(wb_hbm, wb_v)

    wsp = [[wb_v[pl.ds((o * (C * D) + kk) * _L, _L)]
            for kk in range(C * D)] for o in range(D)]
    bsp = [wb_v[pl.ds((C * D * D + o) * _L, _L)] for o in range(D)]
    lane = lax.iota(jnp.int32, _L)

    def group(g, carry):
      item = g * _L + lane
      itemD = item * D
      x1g = plsc.load_gather(x1_v, [item])
      x1gD = x1g * D
      for d in range(D):
        y1d = plsc.load_gather(emb_v, [x1gD + d])
        plsc.store_scatter(y1_v, [itemD + d], y1d)
      acc = [bsp[o] for o in range(D)]
      itemC = item * C
      for j in range(C):
        xj = plsc.load_gather(x_v, [itemC + j])
        xjD = xj * D
        for d in range(D):
          e = plsc.load_gather(emb_v, [xjD + d])
          kk = j * D + d
          for o in range(D):
            acc[o] = acc[o] + wsp[o][kk] * e
      for o in range(D):
        plsc.store_scatter(y_v, [itemD + o], acc[o])
      return carry

    lax.fori_loop(0, groups, group, None)

    pltpu.sync_copy(y_v, y_hbm.at[pl.ds(base * D, bw * D)])
    pltpu.sync_copy(y1_v, y1_hbm.at[pl.ds(base * D, bw * D)])

  return k


def kernel(x, x1, emb, W, b):
  B, C = x.shape
  V, D = emb.shape
  info = plsc.get_sparse_core_info()
  NC, NS = info.num_cores, info.num_subcores
  wb = jnp.repeat(jnp.concatenate([W.reshape(-1), b]), _L)
  k = _make_sc_kernel(B, V, D, C, NC, NS)
  y, y1 = k(x.reshape(-1), x1, emb.reshape(-1), wb)
  return (y.reshape(B, D), y1.reshape(B, D))


# speedup vs baseline: 7.7186x; 1.7062x over previous
import functools

import jax
import jax.numpy as jnp
from jax import lax
from jax.experimental import pallas as pl
from jax.experimental.pallas import tpu as pltpu
from jax.experimental.pallas import tpu_sc as plsc

_L = 16


def _make_sc_kernel(B, V, D, C, NC, NS):
  NW = NC * NS
  bw = B // NW
  groups = bw // _L

  mesh = plsc.VectorSubcoreMesh(core_axis_name="c", subcore_axis_name="s")

  @functools.partial(
      pl.kernel,
      out_type=(
          jax.ShapeDtypeStruct((D * B,), jnp.float32),
          jax.ShapeDtypeStruct((D * B,), jnp.float32),
      ),
      mesh=mesh,
      compiler_params=pltpu.CompilerParams(needs_layout_passes=False),
      scratch_types=[
          pltpu.VMEM((bw * C,), jnp.int32),
          pltpu.VMEM((bw,), jnp.int32),
          pltpu.VMEM((V * D,), jnp.float32),
          pltpu.VMEM(((C * D + 1) * D * _L,), jnp.float32),
          [pltpu.VMEM((bw,), jnp.float32) for _ in range(D)],
          [pltpu.VMEM((bw,), jnp.float32) for _ in range(D)],
      ],
  )
  def k(x_hbm, x1_hbm, emb_hbm, wb_hbm, y_hbm, y1_hbm,
        x_v, x1_v, emb_v, wb_v, y_v, y1_v):
    wid = lax.axis_index("s") * NC + lax.axis_index("c")
    base = wid * bw

    pltpu.sync_copy(x_hbm.at[pl.ds(base * C, bw * C)], x_v)
    pltpu.sync_copy(x1_hbm.at[pl.ds(base, bw)], x1_v)
    pltpu.sync_copy(emb_hbm, emb_v)
    pltpu.sync_copy(wb_hbm, wb_v)

    wsp = [[wb_v[pl.ds((o * (C * D) + kk) * _L, _L)]
            for kk in range(C * D)] for o in range(D)]
    bsp = [wb_v[pl.ds((C * D * D + o) * _L, _L)] for o in range(D)]
    lane = lax.iota(jnp.int32, _L)

    def group(g, carry):
      off = g * _L
      sl = pl.ds(off, _L)
      x1g = x1_v[sl]
      x1gD = x1g * D
      for d in range(D):
        y1_v[d][sl] = plsc.load_gather(emb_v, [x1gD + d])
      acc = [bsp[o] for o in range(D)]
      itemC = (off + lane) * C
      for j in range(C):
        xj = plsc.load_gather(x_v, [itemC + j])
        xjD = xj * D
        for d in range(D):
          e = plsc.load_gather(emb_v, [xjD + d])
          kk = j * D + d
          for o in range(D):
            acc[o] = acc[o] + wsp[o][kk] * e
      for o in range(D):
        y_v[o][sl] = acc[o]
      return carry

    lax.fori_loop(0, groups, group, None)

    for o in range(D):
      pltpu.sync_copy(y_v[o], y_hbm.at[pl.ds(o * B + base, bw)])
      pltpu.sync_copy(y1_v[o], y1_hbm.at[pl.ds(o * B + base, bw)])

  return k


def kernel(x, x1, emb, W, b):
  B, C = x.shape
  V, D = emb.shape
  info = plsc.get_sparse_core_info()
  NC, NS = info.num_cores, info.num_subcores
  wb = jnp.repeat(jnp.concatenate([W.reshape(-1), b]), _L)
  k = _make_sc_kernel(B, V, D, C, NC, NS)
  yT, y1T = k(x.reshape(-1), x1, emb.reshape(-1), wb)
  return (yT.reshape(D, B).T, y1T.reshape(D, B).T)


# speedup vs baseline: 12.9780x; 1.6814x over previous
import functools

import jax
import jax.numpy as jnp
from jax import lax
from jax.experimental import pallas as pl
from jax.experimental.pallas import tpu as pltpu
from jax.experimental.pallas import tpu_sc as plsc

_L = 16
_T = 128


def _make_sc_kernel(B, V, D, C, NC, NS):
  NW = NC * NS
  bw = B // NW
  groups = bw // _L
  P = C

  mesh = plsc.VectorSubcoreMesh(core_axis_name="c", subcore_axis_name="s")

  @functools.partial(
      pl.kernel,
      out_type=(
          jax.ShapeDtypeStruct((P * B,), jnp.float32),
          jax.ShapeDtypeStruct((P * B,), jnp.float32),
      ),
      mesh=mesh,
      compiler_params=pltpu.CompilerParams(needs_layout_passes=False),
      scratch_types=[
          pltpu.VMEM((bw * P,), jnp.int32),
          pltpu.VMEM((bw,), jnp.int32),
          pltpu.VMEM((V * D,), jnp.float32),
          pltpu.VMEM(((C * D + 1) * D * _L,), jnp.float32),
          pltpu.VMEM((bw * P,), jnp.float32),
          pltpu.VMEM((bw * P,), jnp.float32),
      ],
  )
  def k(x_hbm, x1_hbm, emb_hbm, wb_hbm, y_hbm, y1_hbm,
        x_v, x1_v, emb_v, wb_v, y_v, y1_v):
    wid = lax.axis_index("s") * NC + lax.axis_index("c")
    base = wid * bw

    pltpu.sync_copy(x_hbm.at[pl.ds(base * P, bw * P)], x_v)
    pltpu.sync_copy(x1_hbm.at[pl.ds(base, bw)], x1_v)
    pltpu.sync_copy(emb_hbm, emb_v)
    pltpu.sync_copy(wb_hbm, wb_v)

    wsp = [[wb_v[pl.ds((o * (C * D) + kk) * _L, _L)]
            for kk in range(C * D)] for o in range(D)]
    bsp = [wb_v[pl.ds((C * D * D + o) * _L, _L)] for o in range(D)]
    lane = lax.iota(jnp.int32, _L)

    def group(g, carry):
      off = g * _L
      sl = pl.ds(off, _L)
      pbase = (off // _T) * (_T * P) + (off % _T)
      x1g = x1_v[sl]
      x1gD = x1g * D
      for d in range(D):
        y1_v[pl.ds(pbase + d * _T, _L)] = plsc.load_gather(emb_v, [x1gD + d])
      acc = [bsp[o] for o in range(D)]
      pvec = pbase + lane
      for j in range(C):
        xj = plsc.load_gather(x_v, [pvec + j * _T])
        xjD = xj * D
        for d in range(D):
          e = plsc.load_gather(emb_v, [xjD + d])
          kk = j * D + d
          for o in range(D):
            acc[o] = acc[o] + wsp[o][kk] * e
      for o in range(D):
        y_v[pl.ds(pbase + o * _T, _L)] = acc[o]
      return carry

    lax.fori_loop(0, groups, group, None)

    pltpu.sync_copy(y_v, y_hbm.at[pl.ds(base * P, bw * P)])
    pltpu.sync_copy(y1_v, y1_hbm.at[pl.ds(base * P, bw * P)])

  return k


def kernel(x, x1, emb, W, b):
  B, C = x.shape
  V, D = emb.shape
  info = plsc.get_sparse_core_info()
  NC, NS = info.num_cores, info.num_subcores
  wb = jnp.repeat(jnp.concatenate([W.reshape(-1), b]), _L)
  k = _make_sc_kernel(B, V, D, C, NC, NS)
  xp = x.reshape(B // _T, _T, C).transpose(0, 2, 1).reshape(-1)
  yp, y1p = k(xp, x1, emb.reshape(-1), wb)
  unview = lambda p: (
      p.reshape(B // _T, C, _T).transpose(0, 2, 1).reshape(B, C)[:, :D])
  return (unview(yp), unview(y1p))
